# baseline (speedup 1.0000x reference)
import jax
import jax.numpy as jnp
from jax import lax
from jax.experimental import pallas as pl
from jax.experimental.pallas import tpu as pltpu
from jax.experimental.pallas import tpu_sc as plsc

_NC, _NS = 2, 16
_NW = _NC * _NS
_CHUNK = 1568
_PAD = _NW * _CHUNK
_ATM = 50000


def _flags_body(idx_hbm, out_hbm, buf, idx_v):
    wid = lax.axis_index("s") * _NC + lax.axis_index("c")
    base = wid * _CHUNK

    def zero(i, carry):
        buf[pl.ds(pl.multiple_of(16 * i, 16), 16)] = jnp.zeros((16,), jnp.int32)
        return carry

    lax.fori_loop(0, _CHUNK // 16, zero, 0)
    pltpu.sync_copy(idx_hbm, idx_v)
    for k in range(64 // 16):
        v = idx_v[pl.ds(16 * k, 16)]
        v = jnp.where(v < 0, v + _ATM, v)
        r = v - base
        inb = (r >= 0) & (r < _CHUNK)
        r_c = jnp.clip(r, 0, _CHUNK - 1)
        plsc.store_scatter(buf, [r_c], jnp.ones((16,), jnp.int32), mask=inb)
    pltpu.sync_copy(buf, out_hbm.at[pl.ds(base, _CHUNK)])


_flags = pl.kernel(
    _flags_body,
    out_type=jax.ShapeDtypeStruct((_PAD,), jnp.int32),
    mesh=plsc.VectorSubcoreMesh(
        core_axis_name="c", subcore_axis_name="s", num_cores=_NC, num_subcores=_NS
    ),
    scratch_types=[
        pltpu.VMEM((_CHUNK,), jnp.int32),
        pltpu.VMEM((64,), jnp.int32),
    ],
    compiler_params=pltpu.CompilerParams(needs_layout_passes=False),
)


def _copy_body(in_ref, out_ref):
    out_ref[...] = in_ref[...]


_LBLK = 6400


def _copy(pos_t):
    d, b, atm = pos_t.shape
    return pl.pallas_call(
        _copy_body,
        grid=(pl.cdiv(atm, _LBLK),),
        in_specs=[pl.BlockSpec((d, b, _LBLK), lambda i: (0, 0, i))],
        out_specs=pl.BlockSpec((d, b, _LBLK), lambda i: (0, 0, i)),
        out_shape=jax.ShapeDtypeStruct((d, b, atm), pos_t.dtype),
    )(pos_t)


def kernel(pos, idx):
    atm, dim = pos.shape[1], pos.shape[2]
    pos_ct = _copy(jnp.transpose(pos, (2, 0, 1)))
    pos_out = jnp.transpose(pos_ct, (1, 2, 0))
    idx2 = lax.optimization_barrier((pos_ct, idx))[1]
    flags = _flags(idx2)
    msk = jnp.broadcast_to((flags[:atm] != 0)[None, :, None], (1, atm, dim))
    return (pos_out, msk)


# speedup vs baseline: 2.4750x; 2.4750x over previous
import jax
import jax.numpy as jnp
from jax import lax
from jax.experimental import pallas as pl
from jax.experimental.pallas import tpu as pltpu

_ATM = 50000
_LBLK = 6400
_FROWS = 392


def _fused_body(idx_ref, pos_ref, out_ref, flg_ref):
    out_ref[...] = pos_ref[...]

    @pl.when(pl.program_id(0) == 0)
    def _():
        flg_ref[...] = jnp.zeros_like(flg_ref)

        def body(k, carry):
            r = idx_ref[k]
            r = jnp.where(r < 0, r + _ATM, r)
            row = r // 128
            m = lax.broadcasted_iota(jnp.int32, (1, 128), 1) == (r % 128)
            flg_ref[pl.ds(row, 1), :] = flg_ref[pl.ds(row, 1), :] | m
            return carry

        lax.fori_loop(0, idx_ref.shape[0], body, 0)


def _fused(idx, pos_t):
    d, b, atm = pos_t.shape
    return pl.pallas_call(
        _fused_body,
        grid_spec=pltpu.PrefetchScalarGridSpec(
            num_scalar_prefetch=1,
            grid=(pl.cdiv(atm, _LBLK),),
            in_specs=[pl.BlockSpec((d, b, _LBLK), lambda i, idx_ref: (0, 0, i))],
            out_specs=[
                pl.BlockSpec((d, b, _LBLK), lambda i, idx_ref: (0, 0, i)),
                pl.BlockSpec((_FROWS, 128), lambda i, idx_ref: (0, 0)),
            ],
        ),
        out_shape=[
            jax.ShapeDtypeStruct((d, b, atm), pos_t.dtype),
            jax.ShapeDtypeStruct((_FROWS, 128), jnp.bool_),
        ],
    )(idx, pos_t)


def kernel(pos, idx):
    atm, dim = pos.shape[1], pos.shape[2]
    pos_ct, flags = _fused(idx, jnp.transpose(pos, (2, 0, 1)))
    pos_out = jnp.transpose(pos_ct, (1, 2, 0))
    msk = jnp.broadcast_to(flags.reshape(-1)[:atm][None, :, None], (1, atm, dim))
    return (pos_out, msk)
